# baseline (speedup 1.0000x reference)
import jax
import jax.numpy as jnp
from jax.experimental import pallas as pl
from jax.experimental.pallas import tpu as pltpu


def _make_pos_broadcast_kernel(B, H, W, half):
    def _pos_broadcast_kernel(row_ref, col_ref, out_ref, scratch, sem):
        cols_t = col_ref[:W, :].T
        rows_t = row_ref[:H, :].T
        scratch[:half] = jnp.broadcast_to(cols_t[:, None, :], (half, H, W))
        scratch[half:] = jnp.broadcast_to(rows_t[:, :, None], (half, H, W))
        for b in range(B):
            pltpu.make_async_copy(scratch, out_ref.at[b], sem).start()
        for _ in range(B):
            pltpu.make_async_copy(scratch, out_ref.at[0], sem).wait()

    return _pos_broadcast_kernel


def kernel(feature, row_embed, col_embed):
    B, C, H, W = feature.shape
    half = C // 2
    return pl.pallas_call(
        _make_pos_broadcast_kernel(B, H, W, half),
        in_specs=[
            pl.BlockSpec(memory_space=pltpu.MemorySpace.VMEM),
            pl.BlockSpec(memory_space=pltpu.MemorySpace.VMEM),
        ],
        out_specs=pl.BlockSpec(memory_space=pl.ANY),
        out_shape=jax.ShapeDtypeStruct((B, C, H, W), row_embed.dtype),
        scratch_shapes=[
            pltpu.VMEM((C, H, W), row_embed.dtype),
            pltpu.SemaphoreType.DMA,
        ],
    )(row_embed, col_embed)


# speedup vs baseline: 5.5849x; 5.5849x over previous
import jax
import jax.numpy as jnp
from jax.experimental import pallas as pl
from jax.experimental.pallas import tpu as pltpu


def _make_pos_broadcast_kernel(B, H, W, half):
    def _pos_broadcast_kernel(row_ref, col_ref, out_ref, scratch, sem):
        cols = col_ref[:W, :]
        rows = row_ref[:H, :]
        scratch[:, :, :half] = jnp.broadcast_to(cols[None], (H, W, half))
        scratch[:, :, half:] = jnp.broadcast_to(rows[:, None, :], (H, W, half))
        for b in range(B):
            pltpu.make_async_copy(scratch, out_ref.at[b], sem).start()
        for _ in range(B):
            pltpu.make_async_copy(scratch, out_ref.at[0], sem).wait()

    return _pos_broadcast_kernel


def kernel(feature, row_embed, col_embed):
    B, C, H, W = feature.shape
    half = C // 2
    out = pl.pallas_call(
        _make_pos_broadcast_kernel(B, H, W, half),
        in_specs=[
            pl.BlockSpec(memory_space=pltpu.MemorySpace.VMEM),
            pl.BlockSpec(memory_space=pltpu.MemorySpace.VMEM),
        ],
        out_specs=pl.BlockSpec(memory_space=pl.ANY),
        out_shape=jax.ShapeDtypeStruct((B, H, W, C), row_embed.dtype),
        scratch_shapes=[
            pltpu.VMEM((H, W, C), row_embed.dtype),
            pltpu.SemaphoreType.DMA,
        ],
    )(row_embed, col_embed)
    return jnp.transpose(out, (0, 3, 1, 2))


# speedup vs baseline: 6.4723x; 1.1589x over previous
import jax
import jax.numpy as jnp
from jax.experimental import pallas as pl
from jax.experimental.pallas import tpu as pltpu


def _make_pos_broadcast_kernel(B, H, W, half):
    def _pos_broadcast_kernel(row_t_ref, col_t_ref, out_ref, scratch, sem):
        cols = col_t_ref[:, :W].T
        rows = row_t_ref[:, :H].T
        scratch[:, :, :half] = jnp.broadcast_to(cols[None], (H, W, half))
        scratch[:, :, half:] = jnp.broadcast_to(rows[:, None, :], (H, W, half))
        for b in range(B):
            pltpu.make_async_copy(scratch, out_ref.at[b], sem).start()
        for _ in range(B):
            pltpu.make_async_copy(scratch, out_ref.at[0], sem).wait()

    return _pos_broadcast_kernel


def kernel(feature, row_embed, col_embed):
    B, C, H, W = feature.shape
    half = C // 2
    out = pl.pallas_call(
        _make_pos_broadcast_kernel(B, H, W, half),
        in_specs=[
            pl.BlockSpec(memory_space=pltpu.MemorySpace.VMEM),
            pl.BlockSpec(memory_space=pltpu.MemorySpace.VMEM),
        ],
        out_specs=pl.BlockSpec(memory_space=pl.ANY),
        out_shape=jax.ShapeDtypeStruct((B, H, W, C), row_embed.dtype),
        scratch_shapes=[
            pltpu.VMEM((H, W, C), row_embed.dtype),
            pltpu.SemaphoreType.DMA,
        ],
    )(row_embed.T, col_embed.T)
    return jnp.transpose(out, (0, 3, 1, 2))


# speedup vs baseline: 6.8077x; 1.0518x over previous
import jax
import jax.numpy as jnp
from jax.experimental import pallas as pl
from jax.experimental.pallas import tpu as pltpu


def _make_pos_broadcast_kernel(B, H, W, half):
    n_chunks = 4
    hh = H // n_chunks

    def _pos_broadcast_kernel(row_t_ref, col_t_ref, out_ref, scratch, sem):
        cols = col_t_ref[:, :W].T
        rows = row_t_ref[:, :H].T
        for k in range(n_chunks):
            sl = pl.ds(k * hh, hh)
            scratch[sl, :, :half] = jnp.broadcast_to(cols[None], (hh, W, half))
            scratch[sl, :, half:] = jnp.broadcast_to(
                rows[k * hh : (k + 1) * hh, None, :], (hh, W, half)
            )
            for b in range(B):
                pltpu.make_async_copy(
                    scratch.at[sl], out_ref.at[b, sl], sem
                ).start()
        for _ in range(B * n_chunks):
            pltpu.make_async_copy(
                scratch.at[pl.ds(0, hh)], out_ref.at[0, pl.ds(0, hh)], sem
            ).wait()

    return _pos_broadcast_kernel


def kernel(feature, row_embed, col_embed):
    B, C, H, W = feature.shape
    half = C // 2
    out = pl.pallas_call(
        _make_pos_broadcast_kernel(B, H, W, half),
        in_specs=[
            pl.BlockSpec(memory_space=pltpu.MemorySpace.VMEM),
            pl.BlockSpec(memory_space=pltpu.MemorySpace.VMEM),
        ],
        out_specs=pl.BlockSpec(memory_space=pl.ANY),
        out_shape=jax.ShapeDtypeStruct((B, H, W, C), row_embed.dtype),
        scratch_shapes=[
            pltpu.VMEM((H, W, C), row_embed.dtype),
            pltpu.SemaphoreType.DMA,
        ],
    )(row_embed.T, col_embed.T)
    return jnp.transpose(out, (0, 3, 1, 2))
